# baseline (speedup 1.0000x reference)
import functools

import jax
import jax.numpy as jnp
from jax import lax
from jax.experimental import pallas as pl
from jax.experimental.pallas import tpu as pltpu
from jax.experimental.pallas import tpu_sc as plsc

NC = 2
NS = 16
NW = NC * NS
CHUNK = 128
LANES = 16
DEGW = 128


def _fill(ref, rows, cols, value):
    @pl.loop(0, rows)
    def _(r):
        for c0 in range(0, cols, LANES):
            ref[r, pl.ds(c0, LANES)] = jnp.full((LANES,), value, jnp.float32)


def _unpack_dst(pk, j, dbuf):
    for k in range(0, CHUNK, LANES):
        v = pk[j, pl.ds(k, LANES)]
        dbuf[pl.ds(k, LANES)] = lax.shift_right_logical(v, 16)


def _unpack_src(pk, j, sbuf):
    for k in range(0, CHUNK, LANES):
        v = pk[j, pl.ds(k, LANES)]
        sbuf[pl.ds(k, LANES)] = lax.bitwise_and(v, jnp.int32(0xFFFF))


def _make_deg_kernel(n_pad, cpw):
    rps = n_pad // NS
    mesh = plsc.VectorSubcoreMesh(core_axis_name="c", subcore_axis_name="s")

    @functools.partial(
        pl.kernel, mesh=mesh,
        out_type=jax.ShapeDtypeStruct((NC, n_pad, DEGW), jnp.float32),
        scratch_types=[
            pltpu.VMEM((cpw, CHUNK), jnp.int32),
            pltpu.VMEM((CHUNK,), jnp.int32),
            pltpu.VMEM((CHUNK, DEGW), jnp.float32),
            pltpu.VMEM_SHARED((n_pad, DEGW), jnp.float32),
        ],
    )
    def deg_kernel(pk_hbm, out_hbm, pk, dbuf, ones_v, acc):
        c = lax.axis_index("c")
        s = lax.axis_index("s")
        wid = s * NC + c
        _fill(ones_v, CHUNK, DEGW, 1.0)
        for k in range(rps // CHUNK):
            pltpu.sync_copy(ones_v, acc.at[pl.ds(s * rps + k * CHUNK, CHUNK)])
        if rps % CHUNK:
            zr = rps % CHUNK
            pltpu.sync_copy(ones_v.at[pl.ds(0, zr)],
                            acc.at[pl.ds(s * rps + (rps // CHUNK) * CHUNK, zr)])
        pltpu.sync_copy(pk_hbm.at[wid], pk)
        plsc.subcore_barrier()

        @pl.loop(0, cpw)
        def _(j):
            _unpack_dst(pk, j, dbuf)
            pltpu.sync_copy(ones_v, acc.at[dbuf], add=True)

        plsc.subcore_barrier()
        pltpu.sync_copy(acc.at[pl.ds(s * rps, rps)],
                        out_hbm.at[c].at[pl.ds(s * rps, rps)])

    return deg_kernel


def _make_seg_sum_kernel(n_pad, k0, k1, d):
    assert k0 % 4 == 0 and k1 == 0
    hps = k0 // 2
    rps = n_pad // NS
    zr = rps % CHUNK
    mesh = plsc.VectorSubcoreMesh(core_axis_name="c", subcore_axis_name="s")

    @functools.partial(
        pl.kernel, mesh=mesh,
        out_type=jax.ShapeDtypeStruct((1, n_pad, d), jnp.float32),
        scratch_types=[
            pltpu.VMEM((hps, CHUNK), jnp.int32),
            pltpu.VMEM((CHUNK,), jnp.int32),
            pltpu.VMEM((CHUNK,), jnp.int32),
            pltpu.VMEM((CHUNK,), jnp.int32),
            pltpu.VMEM((CHUNK,), jnp.int32),
            pltpu.VMEM((CHUNK, d), jnp.float32),
            pltpu.VMEM((CHUNK, d), jnp.float32),
            pltpu.VMEM_SHARED((n_pad, d), jnp.float32),
            pltpu.SemaphoreType.DMA,
            pltpu.SemaphoreType.DMA,
        ],
    )
    def seg_kernel(h_hbm, pk_hbm, out_hbm,
                   pk, sb0, sb1, db0, db1, rows0, rows1, acc, sem0, sem1):
        c = lax.axis_index("c")
        s = lax.axis_index("s")

        @pl.when(c == 0)
        def _():
            _fill(rows0, CHUNK, d, 0.0)
            for k in range(rps // CHUNK):
                pltpu.sync_copy(rows0,
                                acc.at[pl.ds(s * rps + k * CHUNK, CHUNK)])
            if zr:
                pltpu.sync_copy(
                    rows0.at[pl.ds(0, zr)],
                    acc.at[pl.ds(s * rps + (rps // CHUNK) * CHUNK, zr)])
            plsc.subcore_barrier()

            for half in range(2):
                pltpu.sync_copy(
                    pk_hbm.at[pl.ds(s * k0 + half * hps, hps)], pk)
                _unpack_src(pk, 0, sb0)
                _unpack_dst(pk, 0, db0)
                pltpu.async_copy(h_hbm.at[sb0], rows0, sem0)
                _unpack_src(pk, 1, sb1)
                _unpack_dst(pk, 1, db1)
                pltpu.async_copy(h_hbm.at[sb1], rows1, sem1)

                @pl.loop(0, hps // 2)
                def _(p):
                    j = 2 * p
                    pltpu.make_async_copy(h_hbm.at[sb0], rows0, sem0).wait()
                    pltpu.sync_copy(rows0, acc.at[db0], add=True)

                    @pl.when(j + 2 < hps)
                    def _():
                        _unpack_src(pk, j + 2, sb0)
                        _unpack_dst(pk, j + 2, db0)
                        pltpu.async_copy(h_hbm.at[sb0], rows0, sem0)

                    pltpu.make_async_copy(h_hbm.at[sb1], rows1, sem1).wait()
                    pltpu.sync_copy(rows1, acc.at[db1], add=True)

                    @pl.when(j + 3 < hps)
                    def _():
                        _unpack_src(pk, j + 3, sb1)
                        _unpack_dst(pk, j + 3, db1)
                        pltpu.async_copy(h_hbm.at[sb1], rows1, sem1)

            plsc.subcore_barrier()
            pltpu.sync_copy(acc.at[pl.ds(s * rps, rps)],
                            out_hbm.at[0].at[pl.ds(s * rps, rps)])

    return seg_kernel


def _dinv_block(deg_blk):
    deg = deg_blk[0, :, 0] + deg_blk[1, :, 0] - 1.0
    return lax.rsqrt(deg)


def _tc_scale_matmul(x, w, degp, bn=1000):
    n, f = x.shape
    hid = w.shape[1]

    def body(x_ref, w_ref, deg_ref, o_ref):
        dinv = _dinv_block(deg_ref[...])
        h = jnp.dot(x_ref[...], w_ref[...], preferred_element_type=jnp.float32)
        o_ref[...] = h * dinv[:, None]

    return pl.pallas_call(
        body,
        grid=(n // bn,),
        in_specs=[
            pl.BlockSpec((bn, f), lambda i: (i, 0)),
            pl.BlockSpec((f, hid), lambda i: (0, 0)),
            pl.BlockSpec((NC, bn, DEGW), lambda i: (0, i, 0)),
        ],
        out_specs=pl.BlockSpec((bn, hid), lambda i: (i, 0)),
        out_shape=jax.ShapeDtypeStruct((n, hid), jnp.float32),
    )(x, w, degp)


def _tc_mid(accp, h1p, degp, b1, w2p, bn=1000):
    n, hid = h1p.shape
    c_pad = w2p.shape[1]

    nacc = accp.shape[0]

    def body(a_ref, h_ref, deg_ref, b_ref, w_ref, o_ref):
        dinv = _dinv_block(deg_ref[...])
        a = jnp.sum(a_ref[...], axis=0)
        z = dinv[:, None] * (a + h_ref[...]) + b_ref[...]
        z = jnp.maximum(z, 0.0)
        o_ref[...] = jnp.dot(z, w_ref[...],
                             preferred_element_type=jnp.float32) * dinv[:, None]

    return pl.pallas_call(
        body,
        grid=(n // bn,),
        in_specs=[
            pl.BlockSpec((nacc, bn, hid), lambda i: (0, i, 0)),
            pl.BlockSpec((bn, hid), lambda i: (i, 0)),
            pl.BlockSpec((NC, bn, DEGW), lambda i: (0, i, 0)),
            pl.BlockSpec((1, hid), lambda i: (0, 0)),
            pl.BlockSpec((hid, c_pad), lambda i: (0, 0)),
        ],
        out_specs=pl.BlockSpec((bn, c_pad), lambda i: (i, 0)),
        out_shape=jax.ShapeDtypeStruct((n, c_pad), jnp.float32),
    )(accp, h1p, degp, b1, w2p)


def _tc_final(accp, h2p, degp, b2p, c_real, bn=1000):
    n, c_pad = h2p.shape

    nacc = accp.shape[0]

    def body(a_ref, h_ref, deg_ref, b_ref, o_ref):
        dinv = _dinv_block(deg_ref[...])
        a = jnp.sum(a_ref[...], axis=0)
        z = dinv[:, None] * (a + h_ref[...]) + b_ref[...]
        col = lax.broadcasted_iota(jnp.int32, (bn, c_pad), 1)
        mask = col < c_real
        zm = jnp.where(mask, z, -1e30)
        m = jnp.max(zm, axis=1, keepdims=True)
        e = jnp.where(mask, jnp.exp(zm - m), 0.0)
        lse = jnp.log(jnp.sum(e, axis=1, keepdims=True)) + m
        o_ref[...] = z - lse

    return pl.pallas_call(
        body,
        grid=(n // bn,),
        in_specs=[
            pl.BlockSpec((nacc, bn, c_pad), lambda i: (0, i, 0)),
            pl.BlockSpec((bn, c_pad), lambda i: (i, 0)),
            pl.BlockSpec((NC, bn, DEGW), lambda i: (0, i, 0)),
            pl.BlockSpec((1, c_pad), lambda i: (0, 0)),
        ],
        out_specs=pl.BlockSpec((bn, c_pad), lambda i: (i, 0)),
        out_shape=jax.ShapeDtypeStruct((n, c_pad), jnp.float32),
    )(accp, h2p, degp, b2p)


def kernel(x, edge_index, W1, b1, W2, b2):
    n, f_in = x.shape
    hid = W1.shape[1]
    c_real = W2.shape[1]
    e = edge_index.shape[1]

    cpw = -(-e // (NW * CHUNK))
    cpw += cpw % 2
    e_pad = cpw * NW * CHUNK
    n_pad = -(-(n + 1) // (NS * 8)) * (NS * 8)
    c_pad = 128

    src = edge_index[0]
    dst = edge_index[1]
    pad = e_pad - e
    src_p = jnp.concatenate([src, jnp.zeros((pad,), jnp.int32)])
    dst_p = jnp.concatenate([dst, jnp.full((pad,), n, jnp.int32)])
    pk = src_p | (dst_p << 16)
    pk3 = pk.reshape(NW, cpw, CHUNK)
    ct = e_pad // CHUNK
    k0 = ct // NS
    k1 = 0
    pk2 = pk.reshape(ct, CHUNK)

    w2p = jnp.zeros((hid, c_pad), jnp.float32).at[:, :c_real].set(W2)
    b1r = b1.reshape(1, hid)
    b2p = jnp.zeros((1, c_pad), jnp.float32).at[0, :c_real].set(b2)

    degp = _make_deg_kernel(n_pad, cpw)(pk3)

    h1p = _tc_scale_matmul(x, W1, degp)
    acc1 = _make_seg_sum_kernel(n_pad, k0, k1, hid)(h1p, pk2)
    h2p = _tc_mid(acc1, h1p, degp, b1r, w2p)
    acc2 = _make_seg_sum_kernel(n_pad, k0, k1, c_pad)(h2p, pk2)
    out = _tc_final(acc2, h2p, degp, b2p, c_real)
    return out[:, :c_real]


# speedup vs baseline: 1.0951x; 1.0951x over previous
import functools

import jax
import jax.numpy as jnp
from jax import lax
from jax.experimental import pallas as pl
from jax.experimental.pallas import tpu as pltpu
from jax.experimental.pallas import tpu_sc as plsc

NC = 2
NS = 16
NW = NC * NS
CHUNK = 112
LANES = 16
DEGW = 128


def _fill(ref, rows, cols, value):
    @pl.loop(0, rows)
    def _(r):
        for c0 in range(0, cols, LANES):
            ref[r, pl.ds(c0, LANES)] = jnp.full((LANES,), value, jnp.float32)


def _unpack_dst(pk, j, dbuf):
    for k in range(0, CHUNK, LANES):
        v = pk[j, pl.ds(k, LANES)]
        dbuf[pl.ds(k, LANES)] = lax.shift_right_logical(v, 16)


def _unpack_src(pk, j, sbuf):
    for k in range(0, CHUNK, LANES):
        v = pk[j, pl.ds(k, LANES)]
        sbuf[pl.ds(k, LANES)] = lax.bitwise_and(v, jnp.int32(0xFFFF))


def _make_deg_kernel(n_pad, cpw):
    rps = n_pad // NS
    mesh = plsc.VectorSubcoreMesh(core_axis_name="c", subcore_axis_name="s")

    @functools.partial(
        pl.kernel, mesh=mesh,
        out_type=jax.ShapeDtypeStruct((NC, n_pad, DEGW), jnp.float32),
        scratch_types=[
            pltpu.VMEM((cpw, CHUNK), jnp.int32),
            pltpu.VMEM((CHUNK,), jnp.int32),
            pltpu.VMEM((CHUNK, DEGW), jnp.float32),
            pltpu.VMEM_SHARED((n_pad, DEGW), jnp.float32),
        ],
    )
    def deg_kernel(pk_hbm, out_hbm, pk, dbuf, ones_v, acc):
        c = lax.axis_index("c")
        s = lax.axis_index("s")
        wid = s * NC + c
        _fill(ones_v, CHUNK, DEGW, 1.0)
        for k in range(rps // CHUNK):
            pltpu.sync_copy(ones_v, acc.at[pl.ds(s * rps + k * CHUNK, CHUNK)])
        if rps % CHUNK:
            zr = rps % CHUNK
            pltpu.sync_copy(ones_v.at[pl.ds(0, zr)],
                            acc.at[pl.ds(s * rps + (rps // CHUNK) * CHUNK, zr)])
        pltpu.sync_copy(pk_hbm.at[wid], pk)
        plsc.subcore_barrier()

        @pl.loop(0, cpw)
        def _(j):
            _unpack_dst(pk, j, dbuf)
            pltpu.sync_copy(ones_v, acc.at[dbuf], add=True)

        plsc.subcore_barrier()
        pltpu.sync_copy(acc.at[pl.ds(s * rps, rps)],
                        out_hbm.at[c].at[pl.ds(s * rps, rps)])

    return deg_kernel


def _make_seg_sum_kernel(n_pad, k0, k1, d):
    assert k0 % 8 == 0 and k1 % 8 == 0 and k0 >= k1 >= 8
    rps = n_pad // NS
    zr = rps % CHUNK
    mesh = plsc.VectorSubcoreMesh(core_axis_name="c", subcore_axis_name="s")

    @functools.partial(
        pl.kernel, mesh=mesh,
        out_type=jax.ShapeDtypeStruct((NC, n_pad, d), jnp.float32),
        scratch_types=[
            pltpu.VMEM((k0, CHUNK), jnp.int32),
            pltpu.VMEM((CHUNK,), jnp.int32),
            pltpu.VMEM((CHUNK,), jnp.int32),
            pltpu.VMEM((CHUNK,), jnp.int32),
            pltpu.VMEM((CHUNK,), jnp.int32),
            pltpu.VMEM((CHUNK, d), jnp.float32),
            pltpu.VMEM((CHUNK, d), jnp.float32),
            pltpu.VMEM_SHARED((n_pad, d), jnp.float32),
            pltpu.SemaphoreType.DMA,
            pltpu.SemaphoreType.DMA,
        ],
    )
    def seg_kernel(h_hbm, pk_hbm, out_hbm,
                   pk, sb0, sb1, db0, db1, rows0, rows1, acc, sem0, sem1):
        c = lax.axis_index("c")
        s = lax.axis_index("s")
        _fill(rows0, CHUNK, d, 0.0)
        for k in range(rps // CHUNK):
            pltpu.sync_copy(rows0, acc.at[pl.ds(s * rps + k * CHUNK, CHUNK)])
        if zr:
            pltpu.sync_copy(rows0.at[pl.ds(0, zr)],
                            acc.at[pl.ds(s * rps + (rps // CHUNK) * CHUNK, zr)])
        base = jnp.where(c == 0, s * k0, NS * k0 + s * k1)
        pltpu.sync_copy(pk_hbm.at[pl.ds(base, k0)], pk)
        plsc.subcore_barrier()

        @pl.when(c == 0)
        def _():
            _unpack_src(pk, 0, sb0)
            _unpack_dst(pk, 0, db0)
            pltpu.async_copy(h_hbm.at[sb0], rows0, sem0)
            _unpack_src(pk, 1, sb1)
            _unpack_dst(pk, 1, db1)
            pltpu.async_copy(h_hbm.at[sb1], rows1, sem1)

            @pl.loop(0, k0 // 2)
            def _(p):
                j = 2 * p
                pltpu.make_async_copy(h_hbm.at[sb0], rows0, sem0).wait()
                pltpu.sync_copy(rows0, acc.at[db0], add=True)

                @pl.when(j + 2 < k0)
                def _():
                    _unpack_src(pk, j + 2, sb0)
                    _unpack_dst(pk, j + 2, db0)
                    pltpu.async_copy(h_hbm.at[sb0], rows0, sem0)

                pltpu.make_async_copy(h_hbm.at[sb1], rows1, sem1).wait()
                pltpu.sync_copy(rows1, acc.at[db1], add=True)

                @pl.when(j + 3 < k0)
                def _():
                    _unpack_src(pk, j + 3, sb1)
                    _unpack_dst(pk, j + 3, db1)
                    pltpu.async_copy(h_hbm.at[sb1], rows1, sem1)

        @pl.when(c == 1)
        def _():
            @pl.loop(0, k1)
            def _(j):
                _unpack_src(pk, j, sb0)
                _unpack_dst(pk, j, db0)
                pltpu.async_copy(h_hbm.at[sb0], rows0, sem0).wait()
                pltpu.sync_copy(rows0, acc.at[db0], add=True)

        plsc.subcore_barrier()
        pltpu.sync_copy(acc.at[pl.ds(s * rps, rps)],
                        out_hbm.at[c].at[pl.ds(s * rps, rps)])

    return seg_kernel


def _dinv_block(deg_blk):
    deg = deg_blk[0, :, 0] + deg_blk[1, :, 0] - 1.0
    return lax.rsqrt(deg)


def _tc_scale_matmul(x, w, degp, bn=1000):
    n, f = x.shape
    hid = w.shape[1]

    def body(x_ref, w_ref, deg_ref, o_ref):
        dinv = _dinv_block(deg_ref[...])
        h = jnp.dot(x_ref[...], w_ref[...], preferred_element_type=jnp.float32)
        o_ref[...] = h * dinv[:, None]

    return pl.pallas_call(
        body,
        grid=(n // bn,),
        in_specs=[
            pl.BlockSpec((bn, f), lambda i: (i, 0)),
            pl.BlockSpec((f, hid), lambda i: (0, 0)),
            pl.BlockSpec((NC, bn, DEGW), lambda i: (0, i, 0)),
        ],
        out_specs=pl.BlockSpec((bn, hid), lambda i: (i, 0)),
        out_shape=jax.ShapeDtypeStruct((n, hid), jnp.float32),
    )(x, w, degp)


def _tc_mid(accp, h1p, degp, b1, w2p, bn=1000):
    n, hid = h1p.shape
    c_pad = w2p.shape[1]

    nacc = accp.shape[0]

    def body(a_ref, h_ref, deg_ref, b_ref, w_ref, o_ref):
        dinv = _dinv_block(deg_ref[...])
        a = jnp.sum(a_ref[...], axis=0)
        z = dinv[:, None] * (a + h_ref[...]) + b_ref[...]
        z = jnp.maximum(z, 0.0)
        o_ref[...] = jnp.dot(z, w_ref[...],
                             preferred_element_type=jnp.float32) * dinv[:, None]

    return pl.pallas_call(
        body,
        grid=(n // bn,),
        in_specs=[
            pl.BlockSpec((nacc, bn, hid), lambda i: (0, i, 0)),
            pl.BlockSpec((bn, hid), lambda i: (i, 0)),
            pl.BlockSpec((NC, bn, DEGW), lambda i: (0, i, 0)),
            pl.BlockSpec((1, hid), lambda i: (0, 0)),
            pl.BlockSpec((hid, c_pad), lambda i: (0, 0)),
        ],
        out_specs=pl.BlockSpec((bn, c_pad), lambda i: (i, 0)),
        out_shape=jax.ShapeDtypeStruct((n, c_pad), jnp.float32),
    )(accp, h1p, degp, b1, w2p)


def _tc_final(accp, h2p, degp, b2p, c_real, bn=1000):
    n, c_pad = h2p.shape

    nacc = accp.shape[0]

    def body(a_ref, h_ref, deg_ref, b_ref, o_ref):
        dinv = _dinv_block(deg_ref[...])
        a = jnp.sum(a_ref[...], axis=0)
        z = dinv[:, None] * (a + h_ref[...]) + b_ref[...]
        col = lax.broadcasted_iota(jnp.int32, (bn, c_pad), 1)
        mask = col < c_real
        zm = jnp.where(mask, z, -1e30)
        m = jnp.max(zm, axis=1, keepdims=True)
        e = jnp.where(mask, jnp.exp(zm - m), 0.0)
        lse = jnp.log(jnp.sum(e, axis=1, keepdims=True)) + m
        o_ref[...] = z - lse

    return pl.pallas_call(
        body,
        grid=(n // bn,),
        in_specs=[
            pl.BlockSpec((nacc, bn, c_pad), lambda i: (0, i, 0)),
            pl.BlockSpec((bn, c_pad), lambda i: (i, 0)),
            pl.BlockSpec((NC, bn, DEGW), lambda i: (0, i, 0)),
            pl.BlockSpec((1, c_pad), lambda i: (0, 0)),
        ],
        out_specs=pl.BlockSpec((bn, c_pad), lambda i: (i, 0)),
        out_shape=jax.ShapeDtypeStruct((n, c_pad), jnp.float32),
    )(accp, h2p, degp, b2p)


def kernel(x, edge_index, W1, b1, W2, b2):
    n, f_in = x.shape
    hid = W1.shape[1]
    c_real = W2.shape[1]
    e = edge_index.shape[1]

    cpw = -(-e // (NW * CHUNK))
    cpw = -(-cpw // 4) * 4
    e_pad = cpw * NW * CHUNK
    n_pad = -(-(n + 1) // (NS * 8)) * (NS * 8)
    c_pad = 128

    src = edge_index[0]
    dst = edge_index[1]
    pad = e_pad - e
    src_p = jnp.concatenate([src, jnp.zeros((pad,), jnp.int32)])
    dst_p = jnp.concatenate([dst, jnp.full((pad,), n, jnp.int32)])
    pk = src_p | (dst_p << 16)
    pk3 = pk.reshape(NW, cpw, CHUNK)
    ct = e_pad // CHUNK
    per_core = ct // NS
    k1 = 24
    k0 = per_core - k1
    pk2 = jnp.concatenate(
        [pk.reshape(ct, CHUNK),
         jnp.zeros((k0 - k1, CHUNK), jnp.int32)])

    w2p = jnp.zeros((hid, c_pad), jnp.float32).at[:, :c_real].set(W2)
    b1r = b1.reshape(1, hid)
    b2p = jnp.zeros((1, c_pad), jnp.float32).at[0, :c_real].set(b2)

    degp = _make_deg_kernel(n_pad, cpw)(pk3)

    h1p = _tc_scale_matmul(x, W1, degp)
    acc1 = _make_seg_sum_kernel(n_pad, k0, k1, hid)(h1p, pk2)
    h2p = _tc_mid(acc1, h1p, degp, b1r, w2p)
    acc2 = _make_seg_sum_kernel(n_pad, k0, k1, c_pad)(h2p, pk2)
    out = _tc_final(acc2, h2p, degp, b2p, c_real)
    return out[:, :c_real]
